# initial kernel scaffold (unmeasured)
import jax
import jax.numpy as jnp
from jax import lax
from jax.experimental import pallas as pl
from jax.experimental.pallas import tpu as pltpu


def kernel(
    x,
):
    def body(*refs):
        pass

    out_shape = jax.ShapeDtypeStruct(..., jnp.float32)
    return pl.pallas_call(body, out_shape=out_shape)(...)


# baseline (device time: 18588 ns/iter reference)
import jax
import jax.numpy as jnp
from jax import lax
from jax.experimental import pallas as pl
from jax.experimental.pallas import tpu as pltpu

M = 1024
N = 1024
HALF = 512


def kernel(x):
    def body(x_ref, out_ref, send_buf, recv_buf, send_sem, recv_sem):
        my_x = lax.axis_index("x")
        my_y = lax.axis_index("y")
        peer_y = 1 - my_y

        barrier = pltpu.get_barrier_semaphore()
        pl.semaphore_signal(
            barrier,
            inc=1,
            device_id=(my_x, peer_y),
            device_id_type=pl.DeviceIdType.MESH,
        )
        pl.semaphore_wait(barrier, 1)

        @pl.when(my_y == 0)
        def _():
            send_buf[...] = x_ref[0, :, HALF:].astype(jnp.bfloat16)

        @pl.when(my_y == 1)
        def _():
            send_buf[...] = x_ref[0, :, :HALF].astype(jnp.bfloat16)

        rdma = pltpu.make_async_remote_copy(
            src_ref=send_buf,
            dst_ref=recv_buf,
            send_sem=send_sem,
            recv_sem=recv_sem,
            device_id=(my_x, peer_y),
            device_id_type=pl.DeviceIdType.MESH,
        )
        rdma.start()
        rdma.wait()

        @pl.when(my_y == 0)
        def _():
            out_ref[...] = x_ref[0, :, :HALF] + recv_buf[...].astype(jnp.float32)

        @pl.when(my_y == 1)
        def _():
            out_ref[...] = x_ref[0, :, HALF:] + recv_buf[...].astype(jnp.float32)

    return pl.pallas_call(
        body,
        out_shape=jax.ShapeDtypeStruct((M, HALF), jnp.float32),
        in_specs=[pl.BlockSpec(memory_space=pltpu.VMEM)],
        out_specs=pl.BlockSpec(memory_space=pltpu.VMEM),
        scratch_shapes=[
            pltpu.VMEM((M, HALF), jnp.bfloat16),
            pltpu.VMEM((M, HALF), jnp.bfloat16),
            pltpu.SemaphoreType.DMA,
            pltpu.SemaphoreType.DMA,
        ],
        compiler_params=pltpu.CompilerParams(collective_id=0),
    )(x)


# device time: 16797 ns/iter; 1.1066x vs baseline; 1.1066x over previous
import jax
import jax.numpy as jnp
from jax import lax
from jax.experimental import pallas as pl
from jax.experimental.pallas import tpu as pltpu

M = 1024
N = 1024
HALF = 512
K = 4
CW = HALF // K


def kernel(x):
    def body(
        x_ref,
        out_ref,
        send_y,
        recv_y,
        send_x,
        recv_x,
        sy_send,
        sy_recv,
        sx_send,
        sx_recv,
    ):
        my_x = lax.axis_index("x")
        my_y = lax.axis_index("y")
        peer_x = 1 - my_x
        peer_y = 1 - my_y
        r0 = my_x * HALF
        pr0 = peer_x * HALF

        barrier = pltpu.get_barrier_semaphore()
        for dev in ((my_x, peer_y), (peer_x, my_y)):
            pl.semaphore_signal(
                barrier, inc=1, device_id=dev,
                device_id_type=pl.DeviceIdType.MESH,
            )
        pl.semaphore_wait(barrier, 2)

        rdmas_y = []
        for c in range(K):
            col = c * CW

            @pl.when(my_y == 0)
            def _(col=col, c=c):
                send_y[c] = x_ref[
                    0, pl.ds(r0, HALF), HALF + col:HALF + col + CW
                ].astype(jnp.bfloat16)

            @pl.when(my_y == 1)
            def _(col=col, c=c):
                send_y[c] = x_ref[
                    0, pl.ds(r0, HALF), col:col + CW
                ].astype(jnp.bfloat16)

            rdma = pltpu.make_async_remote_copy(
                src_ref=send_y.at[c],
                dst_ref=recv_y.at[c],
                send_sem=sy_send.at[c],
                recv_sem=sy_recv.at[c],
                device_id=(my_x, peer_y),
                device_id_type=pl.DeviceIdType.MESH,
            )
            rdma.start()
            rdmas_y.append(rdma)

        rdmas_x = []
        for c in range(K):
            col = c * CW
            rdmas_y[c].wait_recv()

            @pl.when(my_y == 0)
            def _(col=col, c=c):
                send_x[c] = (
                    x_ref[0, pl.ds(r0, HALF), col:col + CW].astype(jnp.bfloat16)
                    + recv_y[c]
                )

            @pl.when(my_y == 1)
            def _(col=col, c=c):
                send_x[c] = (
                    x_ref[
                        0, pl.ds(r0, HALF), HALF + col:HALF + col + CW
                    ].astype(jnp.bfloat16)
                    + recv_y[c]
                )

            rdma = pltpu.make_async_remote_copy(
                src_ref=send_x.at[c],
                dst_ref=recv_x.at[c],
                send_sem=sx_send.at[c],
                recv_sem=sx_recv.at[c],
                device_id=(peer_x, my_y),
                device_id_type=pl.DeviceIdType.MESH,
            )
            rdma.start()
            rdmas_x.append(rdma)

            out_ref[pl.ds(r0, HALF), col:col + CW] = send_x[c].astype(
                jnp.float32
            )

        for c in range(K):
            col = c * CW
            rdmas_x[c].wait_recv()
            out_ref[pl.ds(pr0, HALF), col:col + CW] = recv_x[c].astype(
                jnp.float32
            )

        for c in range(K):
            rdmas_y[c].wait_send()
            rdmas_x[c].wait_send()

    return pl.pallas_call(
        body,
        out_shape=jax.ShapeDtypeStruct((M, HALF), jnp.float32),
        in_specs=[pl.BlockSpec(memory_space=pltpu.VMEM)],
        out_specs=pl.BlockSpec(memory_space=pltpu.VMEM),
        scratch_shapes=[
            pltpu.VMEM((K, HALF, CW), jnp.bfloat16),
            pltpu.VMEM((K, HALF, CW), jnp.bfloat16),
            pltpu.VMEM((K, HALF, CW), jnp.bfloat16),
            pltpu.VMEM((K, HALF, CW), jnp.bfloat16),
            pltpu.SemaphoreType.DMA((K,)),
            pltpu.SemaphoreType.DMA((K,)),
            pltpu.SemaphoreType.DMA((K,)),
            pltpu.SemaphoreType.DMA((K,)),
        ],
        compiler_params=pltpu.CompilerParams(collective_id=0),
    )(x)
